# Initial kernel scaffold; baseline (speedup 1.0000x reference)
import jax
import jax.numpy as jnp
from jax.experimental import pallas as pl


def kernel(positions, adj, W0, W1, W2, W3, W4, W5, W6, W7, W8, W9, W10, W11, W12, W13, W14, W15, W16, b0, b1, b2, b3, b4, b5, b6, b7, b8, b9, b10, b11, b12, b13, b14, b15, b16):
    raise NotImplementedError("write your pallas kernel here")


# speedup vs baseline: 1.6247x; 1.6247x over previous
import jax
import jax.numpy as jnp
from jax.experimental import pallas as pl
from jax.experimental.pallas import tpu as pltpu

_N_LAYERS = 17


def _elu(x):
    return jnp.where(x > 0, x, jnp.exp(jnp.minimum(x, 0.0)) - 1.0)


def _mesh_encoder_body(pos_ref, adj_ref, *refs):
    w_refs = refs[:_N_LAYERS]
    b_refs = refs[_N_LAYERS:2 * _N_LAYERS]
    out_ref = refs[2 * _N_LAYERS]

    adj = adj_ref[...]
    norm = jnp.sum(adj, axis=1, keepdims=True)
    x = pos_ref[...]
    for i in range(_N_LAYERS):
        w = w_refs[i][...]
        b = b_refs[i][...]
        support = jnp.dot(x, w, preferred_element_type=jnp.float32)
        side = max(support.shape[1] // 3, 2)
        ns = support[:, :side] / norm
        side1 = jnp.dot(adj, ns, preferred_element_type=jnp.float32)
        support = jnp.concatenate([side1, support[:, side:]], axis=1) + b
        if i < _N_LAYERS - 1:
            x = _elu(support)
        else:
            out_ref[...] = _elu(jnp.max(support, axis=0, keepdims=True))


def kernel(positions, adj, W0, W1, W2, W3, W4, W5, W6, W7, W8, W9, W10, W11, W12, W13, W14, W15, W16, b0, b1, b2, b3, b4, b5, b6, b7, b8, b9, b10, b11, b12, b13, b14, b15, b16):
    ws = [W0, W1, W2, W3, W4, W5, W6, W7, W8, W9, W10, W11, W12, W13, W14, W15, W16]
    bs = [b0, b1, b2, b3, b4, b5, b6, b7, b8, b9, b10, b11, b12, b13, b14, b15, b16]
    bs2d = [b.reshape(1, -1) for b in bs]
    out = pl.pallas_call(
        _mesh_encoder_body,
        out_shape=jax.ShapeDtypeStruct((1, ws[-1].shape[1]), jnp.float32),
        compiler_params=pltpu.CompilerParams(
            vmem_limit_bytes=100 * 1024 * 1024,
        ),
    )(positions, adj, *ws, *bs2d)
    return out.reshape(-1)
